# baseline (speedup 1.0000x reference)
import functools

import jax
import jax.numpy as jnp
from jax import lax
from jax.experimental import pallas as pl
from jax.experimental.pallas import tpu as pltpu
from jax.experimental.pallas import tpu_sc as plsc

_N = 10000
_NP = 10112
_E = 320000
_H = 64
_W = 128
_NC = 2
_NS = 16
_NW = _NC * _NS
_CH = 80
_NIT = 128
_E2 = _NW * _NIT * _CH
_NR = _E2 // _CH
_RPT = _NP // _NS
_CP = [80, 80, 80, 80, 80, 80, 80, 72]


def _sc_layer_body(g_hbm, ea_hbm, src_hbm, dst_hbm, t_hbm,
                   acc_out,
                   src_b, dst_b, ea_v, rows_v, tv,
                   acc_sh, bsem, esem, gsem, ssem):
    c = lax.axis_index("c")
    s = lax.axis_index("s")
    wid = c * _NS + s

    z16 = jnp.zeros((16,), jnp.float32)

    def _zero_row(j, _):
        for cc in range(_W // 16):
            rows_v[0][j, pl.ds(cc * 16, 16)] = z16
        return 0

    lax.fori_loop(0, _CH, _zero_row, 0)
    off = 0
    for n in _CP:
        dsp = pl.ds(s * _RPT + off, n)
        pltpu.sync_copy(rows_v[0].at[pl.ds(0, n)], acc_sh.at[dsp])
        off += n
    pltpu.sync_copy(t_hbm, tv)
    plsc.subcore_barrier()

    tvreg = tv[...]
    row0 = wid * _NIT

    def _issue_blk(blk, bp):
        rr = pl.ds(row0 + blk * 8, 8)
        pltpu.async_copy(src_hbm.at[rr], src_b[bp], bsem[bp])
        pltpu.async_copy(dst_hbm.at[rr], dst_b[bp], bsem[bp])

    def _wait_blk(blk, bp):
        rr = pl.ds(row0 + blk * 8, 8)
        pltpu.make_async_copy(src_hbm.at[rr], src_b[bp], bsem[bp]).wait()
        pltpu.make_async_copy(dst_hbm.at[rr], dst_b[bp], bsem[bp]).wait()

    def _ea_base(it):
        return (row0 + it) * _CH

    def _wait_scatter(it, k, bp):
        pltpu.make_async_copy(rows_v[it % 2], acc_sh.at[dst_b[bp].at[k]],
                              ssem[it % 2]).wait()

    def _compute(r2, u=2):
        @plsc.parallel_loop(0, _CH, unroll=u)
        def _row(j):
            for cc in range(4):
                lane = pl.ds(cc * 16, 16)
                hv = rows_v[r2][j, lane]
                ev = ea_v[r2][j, lane]
                m = jnp.maximum(hv + ev, 0.0) + 1e-7
                w = jnp.exp(m * tvreg)
                rows_v[r2][j, lane] = m * w
                rows_v[r2][j, pl.ds(64 + cc * 16, 16)] = w

    def _gissue(kn, bpn, rn):
        pltpu.async_copy(g_hbm.at[src_b[bpn].at[kn]], rows_v[rn], gsem[rn])

    def _gwait(k, bp, r):
        pltpu.make_async_copy(g_hbm.at[src_b[bp].at[k]], rows_v[r],
                              gsem[r]).wait()

    def _ea_issue(it_t, r):
        pltpu.async_copy(ea_hbm.at[pl.ds(_ea_base(it_t), _CH)], ea_v[r],
                         esem[r])

    def _ea_wait(it_t, r):
        pltpu.make_async_copy(ea_hbm.at[pl.ds(_ea_base(it_t), _CH)],
                              ea_v[r], esem[r]).wait()

    def _step(it_t, ph, blk_t, blk_ph,
              ea_wait_next, sc_wait, g_issue_next, ea_issue2,
              blk_issue, blk_wait, u=2):
        k = ph % 8
        bp = blk_ph % 2
        r = ph % 2
        rn = (ph + 1) % 2
        if blk_issue:
            _issue_blk(blk_t + 1, (blk_ph + 1) % 2)
        if blk_wait:
            _wait_blk(blk_t + 1, (blk_ph + 1) % 2)
        if ea_wait_next:
            _ea_wait(it_t + 1, rn)
        if sc_wait:
            _wait_scatter(ph - 1, (ph - 1) % 8,
                          bp if k != 0 else (blk_ph - 1) % 2)
        if g_issue_next:
            if k == 7:
                _gissue(0, (blk_ph + 1) % 2, rn)
            else:
                _gissue(k + 1, bp, rn)
        _gwait(k, bp, r)
        _compute(r, u)
        pltpu.async_copy(rows_v[r], acc_sh.at[dst_b[bp].at[k]],
                         ssem[r], add=True)
        if ea_issue2:
            _ea_issue(it_t + 2, r)

    _issue_blk(0, 0)
    _ea_issue(0, 0)
    _ea_issue(1, 1)
    _wait_blk(0, 0)
    _ea_wait(0, 0)
    _gissue(0, 0, 0)

    for it in range(8):
        _step(it, it, 0, 0,
              True, it >= 1, True, True, it == 1, it == 7, u=1)

    def _fbody(g_, _):
        it0 = g_ * 16 + 8
        blk0 = g_ * 2 + 1
        for bb in range(16):
            _step(it0 + bb, bb + 8, blk0 + bb // 8, 1 + bb // 8,
                  True, True, True, True,
                  bb % 8 == 1, bb % 8 == 7)
        return 0

    lax.fori_loop(0, 7, _fbody, 0)
    for it in range(120, _NIT):
        _step(it, it, 15, 15,
              it + 1 < _NIT, True, it + 1 < _NIT, it + 2 < _NIT,
              False, False, u=1)
    _wait_scatter(_NIT - 1, 7, 1)
    plsc.subcore_barrier()

    off = 0
    for n in _CP:
        dsp = pl.ds(s * _RPT + off, n)
        pltpu.sync_copy(acc_sh.at[dsp], rows_v[0].at[pl.ds(0, n)])
        pltpu.sync_copy(rows_v[0].at[pl.ds(0, n)], acc_out.at[c, dsp])
        off += n


@jax.jit
def _sc_layer(g, ea, src2d, dst2d, t16):
    mesh = plsc.VectorSubcoreMesh(core_axis_name="c", subcore_axis_name="s")
    f = pl.kernel(
        _sc_layer_body,
        out_type=jax.ShapeDtypeStruct((_NC, _NP, _W), jnp.float32),
        mesh=mesh,
        scratch_types=[
            [pltpu.VMEM((8, _CH), jnp.int32) for _ in range(2)],
            [pltpu.VMEM((8, _CH), jnp.int32) for _ in range(2)],
            [pltpu.VMEM((_CH, _H), jnp.float32) for _ in range(2)],
            [pltpu.VMEM((_CH, _W), jnp.float32) for _ in range(2)],
            pltpu.VMEM((16,), jnp.float32),
            pltpu.VMEM_SHARED((_NP, _W), jnp.float32),
            [pltpu.SemaphoreType.DMA for _ in range(2)],
            [pltpu.SemaphoreType.DMA for _ in range(2)],
            [pltpu.SemaphoreType.DMA for _ in range(2)],
            [pltpu.SemaphoreType.DMA for _ in range(2)],
        ],
    )
    return f(g, ea, src2d, dst2d, t16)


_RB = 632


def _henc_body(x_ref, w_ref, b_ref, o_ref):
    o_ref[...] = jnp.dot(x_ref[...], w_ref[...],
                         preferred_element_type=jnp.float32) + b_ref[...]


def _matmul_bias(x, w, b, rb):
    n = x.shape[0]
    return pl.pallas_call(
        _henc_body,
        grid=(n // rb,),
        in_specs=[
            pl.BlockSpec((rb, x.shape[1]), lambda i: (i, 0)),
            pl.BlockSpec((w.shape[0], w.shape[1]), lambda i: (0, 0)),
            pl.BlockSpec((1, b.shape[1]), lambda i: (0, 0)),
        ],
        out_specs=pl.BlockSpec((rb, w.shape[1]), lambda i: (i, 0)),
        out_shape=jax.ShapeDtypeStruct((n, w.shape[1]), jnp.float32),
    )(x, w, b)


def _ln(z, g, b):
    mu = jnp.mean(z, axis=-1, keepdims=True)
    var = jnp.mean((z - mu) ** 2, axis=-1, keepdims=True)
    return (z - mu) * lax.rsqrt(var + 1e-5) * g + b


def _combine_body(p_ref, g_ref, h_ref,
                  w1_ref, b1_ref, g1_ref, be1_ref, w2_ref, b2_ref,
                  ng_ref, nb_ref, pw_ref, pb_ref,
                  h_out, g_out, *, residual, final):
    p = p_ref[0] + p_ref[1]
    numer = p[:, :_H]
    denom = p[:, _H:]
    out = numer / (denom + 1e-16) + g_ref[:, :_H]
    z = jnp.dot(out, w1_ref[...], preferred_element_type=jnp.float32) + b1_ref[...]
    z = jax.nn.relu(_ln(z, g1_ref[...], be1_ref[...]))
    z = jnp.dot(z, w2_ref[...], preferred_element_type=jnp.float32) + b2_ref[...]
    if residual:
        z = z + h_ref[...]
    nxt = jax.nn.relu(_ln(z, ng_ref[...], nb_ref[...]))
    h_out[...] = z
    if final:
        g_out[...] = jnp.dot(nxt, pw_ref[...],
                             preferred_element_type=jnp.float32) + pb_ref[...]
    else:
        g_out[...] = jnp.concatenate(
            [nxt, jnp.zeros_like(nxt)], axis=1)


def _combine(p, g, h, w1, b1, g1, be1, w2, b2,
             ng, nb, pw, pb, residual, final):
    body = functools.partial(_combine_body, residual=residual, final=final)
    cout = pw.shape[1] if final else _W
    out_shape = [
        jax.ShapeDtypeStruct((_NP, _H), jnp.float32),
        jax.ShapeDtypeStruct((_NP, cout), jnp.float32),
    ]
    full = lambda a: pl.BlockSpec((a.shape[0], a.shape[1]), lambda i: (0, 0))
    res = pl.pallas_call(
        body,
        grid=(_NP // _RB,),
        in_specs=[
            pl.BlockSpec((_NC, _RB, _W), lambda i: (0, i, 0)),
            pl.BlockSpec((_RB, _W), lambda i: (i, 0)),
            pl.BlockSpec((_RB, _H), lambda i: (i, 0)),
            full(w1), full(b1), full(g1), full(be1), full(w2), full(b2),
            full(ng), full(nb), full(pw), full(pb),
        ],
        out_specs=[
            pl.BlockSpec((_RB, _H), lambda i: (i, 0)),
            pl.BlockSpec((_RB, cout), lambda i: (i, 0)),
        ],
        out_shape=out_shape,
    )(p, g, h, w1, b1, g1, be1, w2, b2, ng, nb, pw, pb)
    return res


def kernel(x, edge_index, edge_attr, enc_W, enc_b, eenc_W, eenc_b,
           W1, b1, ln1_g, ln1_b, W2, b2, t, norm_g, norm_b, lin_W, lin_b):
    pad = _E2 - _E
    src_ids = jnp.pad(edge_index[0], (0, pad)).reshape(_NR, _CH)
    dst_ids = jnp.pad(edge_index[1], (0, pad),
                      constant_values=_N).reshape(_NR, _CH)
    eattr = jnp.pad(edge_attr, ((0, pad), (0, 0)))

    h0 = _matmul_bias(x, enc_W, enc_b.reshape(1, -1), 1000)
    h = jnp.pad(h0, ((0, _NP - _N), (0, 0)))
    g = jnp.pad(h0, ((0, _NP - _N), (0, _W - _H)))
    ea = _matmul_bias(eattr, eenc_W, eenc_b.reshape(1, -1), 8192)

    row = lambda a: a.reshape(1, -1)

    for i in range(4):
        t16 = jnp.full((16,), t[i], dtype=jnp.float32)
        p = _sc_layer(g, ea, src_ids, dst_ids, t16)
        final = i == 3
        ni = 0 if final else i + 1
        h, g = _combine(
            p, g, h,
            W1[i], row(b1[i]), row(ln1_g[i]), row(ln1_b[i]),
            W2[i], row(b2[i]),
            row(norm_g[ni]), row(norm_b[ni]),
            lin_W if final else jnp.zeros((_H, _H), jnp.float32),
            row(lin_b) if final else jnp.zeros((1, _H), jnp.float32),
            residual=(i >= 1), final=final,
        )
    return g[:_N]


# speedup vs baseline: 1.9888x; 1.9888x over previous
import functools

import jax
import jax.numpy as jnp
from jax import lax
from jax.experimental import pallas as pl
from jax.experimental.pallas import tpu as pltpu
from jax.experimental.pallas import tpu_sc as plsc

_N = 10000
_NP = 10240
_E = 320000
_H = 64
_W = 128
_NC = 2
_NS = 16
_NW = _NC * _NS
_EPT = _E // _NW
_CH = 80
_NIT = _EPT // _CH
_RPT = _NP // _NS


def _sc_layer_body(g_hbm, ea_hbm, src_hbm, dst_hbm, t_hbm,
                   acc_out,
                   src_v, dst_v, ea_v, rows_v, tv,
                   acc_sh, isem, dsem, gsem, ssem):
    c = lax.axis_index("c")
    s = lax.axis_index("s")
    wid = c * _NS + s

    z16 = jnp.zeros((16,), jnp.float32)

    def _zero_row(j, _):
        for cc in range(_W // 16):
            rows_v[0][j, pl.ds(cc * 16, 16)] = z16
        return 0

    lax.fori_loop(0, _CH, _zero_row, 0)
    for k in range(_RPT // _CH):
        dsp = pl.ds(s * _RPT + k * _CH, _CH)
        pltpu.sync_copy(rows_v[0], acc_sh.at[dsp])
    pltpu.sync_copy(t_hbm, tv)
    plsc.subcore_barrier()

    tvreg = tv[...]

    def _base(it):
        return wid * _EPT + it * _CH

    def _issue_in(it, ph):
        b = pl.ds(_base(it), _CH)
        pltpu.async_copy(src_hbm.at[b], src_v[ph % 2], isem[ph % 2])
        pltpu.async_copy(ea_hbm.at[b], ea_v[ph % 2], isem[ph % 2])
        pltpu.async_copy(dst_hbm.at[b], dst_v[ph % 4], dsem[ph % 4])

    def _wait_in(it, ph):
        b = pl.ds(_base(it), _CH)
        pltpu.make_async_copy(src_hbm.at[b], src_v[ph % 2], isem[ph % 2]).wait()
        pltpu.make_async_copy(ea_hbm.at[b], ea_v[ph % 2], isem[ph % 2]).wait()
        pltpu.make_async_copy(dst_hbm.at[b], dst_v[ph % 4], dsem[ph % 4]).wait()

    def _wait_scatter(ph):
        pltpu.make_async_copy(rows_v[ph % 2], acc_sh.at[dst_v[ph % 4]],
                              ssem[ph % 2]).wait()

    def _compute(r3, r2):
        @plsc.parallel_loop(0, _CH, unroll=2)
        def _row(j):
            for cc in range(4):
                lane = pl.ds(cc * 16, 16)
                hv = rows_v[r3][j, lane]
                ev = ea_v[r2][j, lane]
                m = jnp.maximum(hv + ev, 0.0) + 1e-7
                w = jnp.exp(m * tvreg)
                rows_v[r3][j, lane] = m * w
                rows_v[r3][j, pl.ds(64 + cc * 16, 16)] = w

    def _step(it, ph, wait_next, issue_next, wait_sc1, issue_in2):
        if wait_next:
            _wait_in(it + 1, ph + 1)
        if wait_sc1:
            _wait_scatter(ph - 1)
        if issue_next:
            pltpu.async_copy(g_hbm.at[src_v[(ph + 1) % 2]],
                             rows_v[(ph + 1) % 2], gsem[(ph + 1) % 2])
        pltpu.make_async_copy(g_hbm.at[src_v[ph % 2]],
                              rows_v[ph % 2], gsem[ph % 2]).wait()
        _compute(ph % 2, ph % 2)
        pltpu.async_copy(rows_v[ph % 2], acc_sh.at[dst_v[ph % 4]],
                         ssem[ph % 2], add=True)
        if issue_in2:
            _issue_in(it + 2, ph + 2)

    _issue_in(0, 0)
    _issue_in(1, 1)
    _wait_in(0, 0)
    pltpu.async_copy(g_hbm.at[src_v[0]], rows_v[0], gsem[0])
    _step(0, 0, True, True, False, True)

    def _body(g_, _):
        it0 = g_ * 4 + 1
        for bb in range(4):
            _step(it0 + bb, bb + 1, True, True, True, True)
        return 0

    lax.fori_loop(0, 30, _body, 0)
    for it in range(121, _NIT):
        _step(it, it, it + 1 < _NIT, it + 1 < _NIT, True, it + 2 < _NIT)
    _wait_scatter(_NIT - 1)
    plsc.subcore_barrier()

    for k in range(_RPT // _CH):
        dsp = pl.ds(s * _RPT + k * _CH, _CH)
        pltpu.sync_copy(acc_sh.at[dsp], rows_v[0])
        pltpu.sync_copy(rows_v[0], acc_out.at[c, dsp])


@jax.jit
def _sc_layer(g, ea, src, dst, t16):
    mesh = plsc.VectorSubcoreMesh(core_axis_name="c", subcore_axis_name="s")
    f = pl.kernel(
        _sc_layer_body,
        out_type=jax.ShapeDtypeStruct((_NC, _NP, _W), jnp.float32),
        mesh=mesh,
        scratch_types=[
            [pltpu.VMEM((_CH,), jnp.int32) for _ in range(2)],
            [pltpu.VMEM((_CH,), jnp.int32) for _ in range(4)],
            [pltpu.VMEM((_CH, _H), jnp.float32) for _ in range(2)],
            [pltpu.VMEM((_CH, _W), jnp.float32) for _ in range(2)],
            pltpu.VMEM((16,), jnp.float32),
            pltpu.VMEM_SHARED((_NP, _W), jnp.float32),
            [pltpu.SemaphoreType.DMA for _ in range(2)],
            [pltpu.SemaphoreType.DMA for _ in range(4)],
            [pltpu.SemaphoreType.DMA for _ in range(2)],
            [pltpu.SemaphoreType.DMA for _ in range(2)],
        ],
    )
    return f(g, ea, src, dst, t16)


_RB = 1024


def _henc_body(x_ref, w_ref, b_ref, o_ref):
    o_ref[...] = jnp.dot(x_ref[...], w_ref[...],
                         preferred_element_type=jnp.float32) + b_ref[...]


def _matmul_bias(x, w, b, rb):
    n = x.shape[0]
    return pl.pallas_call(
        _henc_body,
        grid=(n // rb,),
        in_specs=[
            pl.BlockSpec((rb, x.shape[1]), lambda i: (i, 0)),
            pl.BlockSpec((w.shape[0], w.shape[1]), lambda i: (0, 0)),
            pl.BlockSpec((1, b.shape[1]), lambda i: (0, 0)),
        ],
        out_specs=pl.BlockSpec((rb, w.shape[1]), lambda i: (i, 0)),
        out_shape=jax.ShapeDtypeStruct((n, w.shape[1]), jnp.float32),
    )(x, w, b)


def _ln(z, g, b):
    mu = jnp.mean(z, axis=-1, keepdims=True)
    var = jnp.mean((z - mu) ** 2, axis=-1, keepdims=True)
    return (z - mu) * lax.rsqrt(var + 1e-5) * g + b


def _combine_body(p_ref, g_ref, h_ref,
                  w1_ref, b1_ref, g1_ref, be1_ref, w2_ref, b2_ref,
                  ng_ref, nb_ref, pw_ref, pb_ref,
                  h_out, g_out, *, residual, final):
    p = p_ref[0] + p_ref[1]
    numer = p[:, :_H]
    denom = p[:, _H:]
    out = numer / (denom + 1e-16) + g_ref[:, :_H]
    z = jnp.dot(out, w1_ref[...], preferred_element_type=jnp.float32) + b1_ref[...]
    z = jax.nn.relu(_ln(z, g1_ref[...], be1_ref[...]))
    z = jnp.dot(z, w2_ref[...], preferred_element_type=jnp.float32) + b2_ref[...]
    if residual:
        z = z + h_ref[...]
    nxt = jax.nn.relu(_ln(z, ng_ref[...], nb_ref[...]))
    h_out[...] = z
    if final:
        g_out[...] = jnp.dot(nxt, pw_ref[...],
                             preferred_element_type=jnp.float32) + pb_ref[...]
    else:
        g_out[...] = jnp.concatenate(
            [nxt, jnp.zeros_like(nxt)], axis=1)


def _combine(p, g, h, w1, b1, g1, be1, w2, b2,
             ng, nb, pw, pb, residual, final):
    body = functools.partial(_combine_body, residual=residual, final=final)
    cout = pw.shape[1] if final else _W
    out_shape = [
        jax.ShapeDtypeStruct((_NP, _H), jnp.float32),
        jax.ShapeDtypeStruct((_NP, cout), jnp.float32),
    ]
    full = lambda a: pl.BlockSpec((a.shape[0], a.shape[1]), lambda i: (0, 0))
    res = pl.pallas_call(
        body,
        grid=(_NP // _RB,),
        in_specs=[
            pl.BlockSpec((_NC, _RB, _W), lambda i: (0, i, 0)),
            pl.BlockSpec((_RB, _W), lambda i: (i, 0)),
            pl.BlockSpec((_RB, _H), lambda i: (i, 0)),
            full(w1), full(b1), full(g1), full(be1), full(w2), full(b2),
            full(ng), full(nb), full(pw), full(pb),
        ],
        out_specs=[
            pl.BlockSpec((_RB, _H), lambda i: (i, 0)),
            pl.BlockSpec((_RB, cout), lambda i: (i, 0)),
        ],
        out_shape=out_shape,
    )(p, g, h, w1, b1, g1, be1, w2, b2, ng, nb, pw, pb)
    return res


def kernel(x, edge_index, edge_attr, enc_W, enc_b, eenc_W, eenc_b,
           W1, b1, ln1_g, ln1_b, W2, b2, t, norm_g, norm_b, lin_W, lin_b):
    src = edge_index[0]
    dst = edge_index[1]

    h0 = _matmul_bias(x, enc_W, enc_b.reshape(1, -1), 1000)
    h = jnp.pad(h0, ((0, _NP - _N), (0, 0)))
    g = jnp.pad(h0, ((0, _NP - _N), (0, _W - _H)))
    ea = _matmul_bias(edge_attr, eenc_W, eenc_b.reshape(1, -1), 8000)

    row = lambda a: a.reshape(1, -1)

    for i in range(4):
        t16 = jnp.full((16,), t[i], dtype=jnp.float32)
        p = _sc_layer(g, ea, src, dst, t16)
        final = i == 3
        ni = 0 if final else i + 1
        h, g = _combine(
            p, g, h,
            W1[i], row(b1[i]), row(ln1_g[i]), row(ln1_b[i]),
            W2[i], row(b2[i]),
            row(norm_g[ni]), row(norm_b[ni]),
            lin_W if final else jnp.zeros((_H, _H), jnp.float32),
            row(lin_b) if final else jnp.zeros((1, _H), jnp.float32),
            residual=(i >= 1), final=final,
        )
    return g[:_N]
